# initial kernel scaffold (unmeasured)
import jax
import jax.numpy as jnp
from jax import lax
from jax.experimental import pallas as pl
from jax.experimental.pallas import tpu as pltpu

N_DEV = 8
SQ = 1024
D = 1024
HQ = 8
DH = 128
BLK = SQ // N_DEV
BAND = 384
WIN = 128
HALO = 128
SCALE = 0.08838834764831843


def kernel(x, Wq, K_ext, V_ext, Wo):
    x2 = x.reshape(SQ, D)
    k3 = K_ext.reshape(K_ext.shape[1], HQ, DH)
    v3 = V_ext.reshape(V_ext.shape[1], HQ, DH)

    def body(x_ref, wq_ref, k_ref, v_ref, wo_ref, out_ref,
             q_scr, ctx_scr, ctx_slice, khalo, vhalo, stage,
             halo_send, halo_recv, scat_send, scat_recv,
             ring_send, ring_recv):
        pos = lax.axis_index("i")
        right = (pos + 1) % N_DEV

        @pl.when(pos == 1)
        def _():
            k_rdma = pltpu.make_async_remote_copy(
                src_ref=k_ref.at[pl.ds(0, HALO)], dst_ref=khalo,
                send_sem=halo_send.at[0], recv_sem=halo_recv.at[0],
                device_id=(0,), device_id_type=pl.DeviceIdType.MESH)
            v_rdma = pltpu.make_async_remote_copy(
                src_ref=v_ref.at[pl.ds(0, HALO)], dst_ref=vhalo,
                send_sem=halo_send.at[1], recv_sem=halo_recv.at[1],
                device_id=(0,), device_id_type=pl.DeviceIdType.MESH)
            k_rdma.start()
            v_rdma.start()
            k_rdma.wait_send()
            v_rdma.wait_send()

        @pl.when(pos == 0)
        def _():
            k_wait = pltpu.make_async_remote_copy(
                src_ref=k_ref.at[pl.ds(0, HALO)], dst_ref=khalo,
                send_sem=halo_send.at[0], recv_sem=halo_recv.at[0],
                device_id=(1,), device_id_type=pl.DeviceIdType.MESH)
            v_wait = pltpu.make_async_remote_copy(
                src_ref=v_ref.at[pl.ds(0, HALO)], dst_ref=vhalo,
                send_sem=halo_send.at[1], recv_sem=halo_recv.at[1],
                device_id=(1,), device_id_type=pl.DeviceIdType.MESH)

            q_scr[...] = jnp.dot(x_ref[...], wq_ref[...],
                                 preferred_element_type=jnp.float32)
            k_wait.wait_recv()
            v_wait.wait_recv()

            for qb in range(N_DEV):
                s = max(0, BLK * qb - WIN)
                for h in range(HQ):
                    qh = q_scr[pl.ds(qb * BLK, BLK), pl.ds(h * DH, DH)]
                    if qb < N_DEV - 1:
                        kb = k_ref[pl.ds(s, BAND), h, :]
                        vb = v_ref[pl.ds(s, BAND), h, :]
                    else:
                        kb = jnp.concatenate(
                            [k_ref[pl.ds(s, BAND - HALO), h, :],
                             khalo[:, h, :]], axis=0)
                        vb = jnp.concatenate(
                            [v_ref[pl.ds(s, BAND - HALO), h, :],
                             vhalo[:, h, :]], axis=0)
                    scores = lax.dot_general(
                        qh, kb, (((1,), (1,)), ((), ())),
                        preferred_element_type=jnp.float32) * SCALE
                    qi = qb * BLK + lax.broadcasted_iota(
                        jnp.int32, (BLK, BAND), 0)
                    ki = s + lax.broadcasted_iota(jnp.int32, (BLK, BAND), 1)
                    scores = jnp.where(jnp.abs(qi - ki) <= WIN, scores, -1e9)
                    m = jnp.max(scores, axis=1, keepdims=True)
                    w = jnp.exp(scores - m)
                    w = w / jnp.sum(w, axis=1, keepdims=True)
                    ctx_scr[pl.ds(qb * BLK, BLK), pl.ds(h * DH, DH)] = (
                        jnp.dot(w, vb, preferred_element_type=jnp.float32))

            rdmas = []
            for d in range(1, N_DEV):
                r = pltpu.make_async_remote_copy(
                    src_ref=ctx_scr.at[pl.ds(d * BLK, BLK), :],
                    dst_ref=ctx_slice,
                    send_sem=scat_send.at[d - 1], recv_sem=scat_recv,
                    device_id=(d,), device_id_type=pl.DeviceIdType.MESH)
                r.start()
                rdmas.append(r)
            ctx_slice[...] = ctx_scr[pl.ds(0, BLK), :]
            for r in rdmas:
                r.wait_send()

        @pl.when(pos != 0)
        def _():
            scat_wait = pltpu.make_async_remote_copy(
                src_ref=ctx_slice, dst_ref=ctx_slice,
                send_sem=scat_send.at[0], recv_sem=scat_recv,
                device_id=(0,), device_id_type=pl.DeviceIdType.MESH)
            scat_wait.wait_recv()

        out_slice = jnp.dot(ctx_slice[...], wo_ref[...],
                            preferred_element_type=jnp.float32)
        out_ref[pl.ds(pos * BLK, BLK), :] = out_slice
        stage[0] = out_slice

        for h in range(N_DEV - 1):
            rdma = pltpu.make_async_remote_copy(
                src_ref=stage.at[h], dst_ref=stage.at[h + 1],
                send_sem=ring_send.at[h], recv_sem=ring_recv.at[h],
                device_id=(right,), device_id_type=pl.DeviceIdType.MESH)
            rdma.start()
            rdma.wait()
            origin = (pos - h - 1) % N_DEV
            out_ref[pl.ds(origin * BLK, BLK), :] = stage[h + 1]

    out = pl.pallas_call(
        body,
        out_shape=jax.ShapeDtypeStruct((SQ, D), jnp.float32),
        in_specs=[pl.BlockSpec(memory_space=pltpu.VMEM)] * 5,
        out_specs=pl.BlockSpec(memory_space=pltpu.VMEM),
        scratch_shapes=[
            pltpu.VMEM((SQ, D), jnp.float32),
            pltpu.VMEM((SQ, D), jnp.float32),
            pltpu.VMEM((BLK, D), jnp.float32),
            pltpu.VMEM((HALO, HQ, DH), jnp.float32),
            pltpu.VMEM((HALO, HQ, DH), jnp.float32),
            pltpu.VMEM((N_DEV, BLK, D), jnp.float32),
            pltpu.SemaphoreType.DMA((2,)),
            pltpu.SemaphoreType.DMA((2,)),
            pltpu.SemaphoreType.DMA((N_DEV - 1,)),
            pltpu.SemaphoreType.DMA,
            pltpu.SemaphoreType.DMA((N_DEV - 1,)),
            pltpu.SemaphoreType.DMA((N_DEV - 1,)),
        ],
        compiler_params=pltpu.CompilerParams(collective_id=0),
    )(x2, Wq, k3, v3, Wo)
    return out.reshape(1, SQ, D)


# baseline (device time: 134284 ns/iter reference)
import jax
import jax.numpy as jnp
from jax import lax
from jax.experimental import pallas as pl
from jax.experimental.pallas import tpu as pltpu

N_DEV = 8
SQ = 1024
D = 1024
HQ = 8
DH = 128
BLK = SQ // N_DEV
BAND = 384
WIN = 128
HALO = 128
SCALE = 0.08838834764831843


def kernel(x, Wq, K_ext, V_ext, Wo):
    x2 = x.reshape(SQ, D)
    k3 = K_ext.reshape(K_ext.shape[1], HQ, DH)
    v3 = V_ext.reshape(V_ext.shape[1], HQ, DH)

    def body(x_ref, wq_ref, k_ref, v_ref, wo_ref, out_ref,
             q_scr, ctx_scr, ctx_slice, khalo, vhalo, stage,
             halo_send, halo_recv, scat_send, scat_recv,
             ring_send, ring_recv):
        pos = lax.axis_index("i")
        right = (pos + 1) % N_DEV

        @pl.when(pos == 1)
        def _():
            k_rdma = pltpu.make_async_remote_copy(
                src_ref=k_ref.at[pl.ds(0, HALO)], dst_ref=khalo,
                send_sem=halo_send.at[0], recv_sem=halo_recv.at[0],
                device_id=(0,), device_id_type=pl.DeviceIdType.MESH)
            v_rdma = pltpu.make_async_remote_copy(
                src_ref=v_ref.at[pl.ds(0, HALO)], dst_ref=vhalo,
                send_sem=halo_send.at[1], recv_sem=halo_recv.at[1],
                device_id=(0,), device_id_type=pl.DeviceIdType.MESH)
            k_rdma.start()
            v_rdma.start()
            k_rdma.wait_send()
            v_rdma.wait_send()

        @pl.when(pos == 0)
        def _():
            k_wait = pltpu.make_async_remote_copy(
                src_ref=k_ref.at[pl.ds(0, HALO)], dst_ref=khalo,
                send_sem=halo_send.at[0], recv_sem=halo_recv.at[0],
                device_id=(1,), device_id_type=pl.DeviceIdType.MESH)
            v_wait = pltpu.make_async_remote_copy(
                src_ref=v_ref.at[pl.ds(0, HALO)], dst_ref=vhalo,
                send_sem=halo_send.at[1], recv_sem=halo_recv.at[1],
                device_id=(1,), device_id_type=pl.DeviceIdType.MESH)

            q_scr[...] = jnp.dot(x_ref[...], wq_ref[...],
                                 preferred_element_type=jnp.float32)
            k_wait.wait_recv()
            v_wait.wait_recv()

            for qb in range(N_DEV):
                s = max(0, BLK * qb - WIN)
                for h in range(HQ):
                    qh = q_scr[pl.ds(qb * BLK, BLK), pl.ds(h * DH, DH)]
                    if qb < N_DEV - 1:
                        kb = k_ref[pl.ds(s, BAND), h, :]
                        vb = v_ref[pl.ds(s, BAND), h, :]
                    else:
                        kb = jnp.concatenate(
                            [k_ref[pl.ds(s, BAND - HALO), h, :],
                             khalo[:, h, :]], axis=0)
                        vb = jnp.concatenate(
                            [v_ref[pl.ds(s, BAND - HALO), h, :],
                             vhalo[:, h, :]], axis=0)
                    scores = lax.dot_general(
                        qh, kb, (((1,), (1,)), ((), ())),
                        preferred_element_type=jnp.float32) * SCALE
                    qi = qb * BLK + lax.broadcasted_iota(
                        jnp.int32, (BLK, BAND), 0)
                    ki = s + lax.broadcasted_iota(jnp.int32, (BLK, BAND), 1)
                    scores = jnp.where(jnp.abs(qi - ki) <= WIN, scores, -1e9)
                    m = jnp.max(scores, axis=1, keepdims=True)
                    w = jnp.exp(scores - m)
                    w = w / jnp.sum(w, axis=1, keepdims=True)
                    ctx_scr[pl.ds(qb * BLK, BLK), pl.ds(h * DH, DH)] = (
                        jnp.dot(w, vb, preferred_element_type=jnp.float32))

            rdmas = []
            for d in range(1, N_DEV):
                r = pltpu.make_async_remote_copy(
                    src_ref=ctx_scr.at[pl.ds(d * BLK, BLK), :],
                    dst_ref=ctx_slice,
                    send_sem=scat_send.at[d - 1], recv_sem=scat_recv,
                    device_id=(d,), device_id_type=pl.DeviceIdType.MESH)
                r.start()
                rdmas.append(r)
            ctx_slice[...] = ctx_scr[pl.ds(0, BLK), :]
            for r in rdmas:
                r.wait_send()

        @pl.when(pos != 0)
        def _():
            scat_wait = pltpu.make_async_remote_copy(
                src_ref=ctx_slice, dst_ref=ctx_slice,
                send_sem=scat_send.at[0], recv_sem=scat_recv,
                device_id=(0,), device_id_type=pl.DeviceIdType.MESH)
            scat_wait.wait_recv()

        out_slice = jnp.dot(ctx_slice[...], wo_ref[...],
                            preferred_element_type=jnp.float32)
        out_ref[pl.ds(pos * BLK, BLK), :] = out_slice
        stage[0] = out_slice

        for h in range(N_DEV - 1):
            rdma = pltpu.make_async_remote_copy(
                src_ref=stage.at[h], dst_ref=stage.at[h + 1],
                send_sem=ring_send.at[h], recv_sem=ring_recv.at[h],
                device_id=(right,), device_id_type=pl.DeviceIdType.MESH)
            rdma.start()
            rdma.wait()
            origin = (pos - h - 1) % N_DEV
            out_ref[pl.ds(origin * BLK, BLK), :] = stage[h + 1]

    out = pl.pallas_call(
        body,
        out_shape=jax.ShapeDtypeStruct((SQ, D), jnp.float32),
        in_specs=[pl.BlockSpec(memory_space=pltpu.VMEM)] * 5,
        out_specs=pl.BlockSpec(memory_space=pltpu.VMEM),
        scratch_shapes=[
            pltpu.VMEM((SQ, D), jnp.float32),
            pltpu.VMEM((SQ, D), jnp.float32),
            pltpu.VMEM((BLK, D), jnp.float32),
            pltpu.VMEM((HALO, HQ, DH), jnp.float32),
            pltpu.VMEM((HALO, HQ, DH), jnp.float32),
            pltpu.VMEM((N_DEV, BLK, D), jnp.float32),
            pltpu.SemaphoreType.DMA((2,)),
            pltpu.SemaphoreType.DMA((2,)),
            pltpu.SemaphoreType.DMA((N_DEV - 1,)),
            pltpu.SemaphoreType.DMA,
            pltpu.SemaphoreType.DMA((N_DEV - 1,)),
            pltpu.SemaphoreType.DMA((N_DEV - 1,)),
        ],
    )(x2, Wq, k3, v3, Wo)
    return out.reshape(1, SQ, D)


# device time: 76082 ns/iter; 1.7650x vs baseline; 1.7650x over previous
import jax
import jax.numpy as jnp
from jax import lax
from jax.experimental import pallas as pl
from jax.experimental.pallas import tpu as pltpu

N_DEV = 8
SQ = 1024
D = 1024
HQ = 8
DH = 128
BLK = SQ // N_DEV
BAND = 384
WIN = 128
HALO = 128
SCALE = 0.08838834764831843
N_FWD = 4
N_BWD = 3


def kernel(x, Wq, K_ext, V_ext, Wo):
    x2 = x.reshape(SQ, D)
    k2 = K_ext.reshape(K_ext.shape[1], HQ * DH)
    v2 = V_ext.reshape(V_ext.shape[1], HQ * DH)

    def body(x_ref, wq_ref, k_ref, v_ref, wo_ref, out_ref,
             ctx_scr, ctx_slice, khalo, vhalo, fstage, bstage,
             halo_send, halo_recv, scat_send, scat_recv,
             fsend, frecv, bsend, brecv):
        pos = lax.axis_index("i")
        right = (pos + 1) % N_DEV
        left = (pos - 1) % N_DEV

        @pl.when(pos == 1)
        def _():
            k_rdma = pltpu.make_async_remote_copy(
                src_ref=k_ref.at[pl.ds(0, HALO), :], dst_ref=khalo,
                send_sem=halo_send.at[0], recv_sem=halo_recv.at[0],
                device_id=(0,), device_id_type=pl.DeviceIdType.MESH)
            v_rdma = pltpu.make_async_remote_copy(
                src_ref=v_ref.at[pl.ds(0, HALO), :], dst_ref=vhalo,
                send_sem=halo_send.at[1], recv_sem=halo_recv.at[1],
                device_id=(0,), device_id_type=pl.DeviceIdType.MESH)
            k_rdma.start()
            v_rdma.start()
            k_rdma.wait_send()
            v_rdma.wait_send()

        @pl.when(pos == 0)
        def _():
            k_wait = pltpu.make_async_remote_copy(
                src_ref=k_ref.at[pl.ds(0, HALO), :], dst_ref=khalo,
                send_sem=halo_send.at[0], recv_sem=halo_recv.at[0],
                device_id=(1,), device_id_type=pl.DeviceIdType.MESH)
            v_wait = pltpu.make_async_remote_copy(
                src_ref=v_ref.at[pl.ds(0, HALO), :], dst_ref=vhalo,
                send_sem=halo_send.at[1], recv_sem=halo_recv.at[1],
                device_id=(1,), device_id_type=pl.DeviceIdType.MESH)

            rdmas = []
            for qb in list(range(1, N_DEV)) + [0]:
                s = max(0, BLK * qb - WIN)
                q_blk = jnp.dot(x_ref[pl.ds(qb * BLK, BLK), :], wq_ref[...],
                                preferred_element_type=jnp.float32)
                if qb < N_DEV - 1:
                    kband = k_ref[pl.ds(s, BAND), :]
                    vband = v_ref[pl.ds(s, BAND), :]
                else:
                    k_wait.wait_recv()
                    v_wait.wait_recv()
                    kband = jnp.concatenate(
                        [k_ref[pl.ds(s, BAND - HALO), :], khalo[...]], axis=0)
                    vband = jnp.concatenate(
                        [v_ref[pl.ds(s, BAND - HALO), :], vhalo[...]], axis=0)
                qi = qb * BLK + lax.broadcasted_iota(jnp.int32, (BLK, BAND), 0)
                ki = s + lax.broadcasted_iota(jnp.int32, (BLK, BAND), 1)
                mask = jnp.abs(qi - ki) <= WIN
                for h in range(HQ):
                    qh = q_blk[:, h * DH:(h + 1) * DH]
                    kb = kband[:, h * DH:(h + 1) * DH]
                    vb = vband[:, h * DH:(h + 1) * DH]
                    scores = lax.dot_general(
                        qh, kb, (((1,), (1,)), ((), ())),
                        preferred_element_type=jnp.float32) * SCALE
                    scores = jnp.where(mask, scores, -1e9)
                    m = jnp.max(scores, axis=1, keepdims=True)
                    w = jnp.exp(scores - m)
                    w = w / jnp.sum(w, axis=1, keepdims=True)
                    ctx_scr[pl.ds(qb * BLK, BLK), pl.ds(h * DH, DH)] = (
                        jnp.dot(w, vb, preferred_element_type=jnp.float32))
                if qb != 0:
                    r = pltpu.make_async_remote_copy(
                        src_ref=ctx_scr.at[pl.ds(qb * BLK, BLK), :],
                        dst_ref=ctx_slice,
                        send_sem=scat_send.at[qb - 1], recv_sem=scat_recv,
                        device_id=(qb,), device_id_type=pl.DeviceIdType.MESH)
                    r.start()
                    rdmas.append(r)
            ctx_slice[...] = ctx_scr[pl.ds(0, BLK), :]
            for r in rdmas:
                r.wait_send()

        @pl.when(pos != 0)
        def _():
            scat_wait = pltpu.make_async_remote_copy(
                src_ref=ctx_slice, dst_ref=ctx_slice,
                send_sem=scat_send.at[0], recv_sem=scat_recv,
                device_id=(0,), device_id_type=pl.DeviceIdType.MESH)
            scat_wait.wait_recv()

        out_slice = jnp.dot(ctx_slice[...], wo_ref[...],
                            preferred_element_type=jnp.float32)
        out_ref[pl.ds(pos * BLK, BLK), :] = out_slice
        fstage[0] = out_slice
        bstage[0] = out_slice

        f_rdma = [
            pltpu.make_async_remote_copy(
                src_ref=fstage.at[h], dst_ref=fstage.at[h + 1],
                send_sem=fsend.at[h], recv_sem=frecv.at[h],
                device_id=(right,), device_id_type=pl.DeviceIdType.MESH)
            for h in range(N_FWD)]
        b_rdma = [
            pltpu.make_async_remote_copy(
                src_ref=bstage.at[h], dst_ref=bstage.at[h + 1],
                send_sem=bsend.at[h], recv_sem=brecv.at[h],
                device_id=(left,), device_id_type=pl.DeviceIdType.MESH)
            for h in range(N_BWD)]
        f_rdma[0].start()
        b_rdma[0].start()
        for h in range(N_FWD):
            f_rdma[h].wait_recv()
            if h + 1 < N_FWD:
                f_rdma[h + 1].start()
            if h < N_BWD:
                b_rdma[h].wait_recv()
                if h + 1 < N_BWD:
                    b_rdma[h + 1].start()
            out_ref[pl.ds(((pos - h - 1) % N_DEV) * BLK, BLK), :] = (
                fstage[h + 1])
            if h < N_BWD:
                out_ref[pl.ds(((pos + h + 1) % N_DEV) * BLK, BLK), :] = (
                    bstage[h + 1])
        for r in f_rdma:
            r.wait_send()
        for r in b_rdma:
            r.wait_send()

    out = pl.pallas_call(
        body,
        out_shape=jax.ShapeDtypeStruct((SQ, D), jnp.float32),
        in_specs=[pl.BlockSpec(memory_space=pltpu.VMEM)] * 5,
        out_specs=pl.BlockSpec(memory_space=pltpu.VMEM),
        scratch_shapes=[
            pltpu.VMEM((SQ, D), jnp.float32),
            pltpu.VMEM((BLK, D), jnp.float32),
            pltpu.VMEM((HALO, HQ * DH), jnp.float32),
            pltpu.VMEM((HALO, HQ * DH), jnp.float32),
            pltpu.VMEM((N_FWD + 1, BLK, D), jnp.float32),
            pltpu.VMEM((N_BWD + 1, BLK, D), jnp.float32),
            pltpu.SemaphoreType.DMA((2,)),
            pltpu.SemaphoreType.DMA((2,)),
            pltpu.SemaphoreType.DMA((N_DEV - 1,)),
            pltpu.SemaphoreType.DMA,
            pltpu.SemaphoreType.DMA((N_FWD,)),
            pltpu.SemaphoreType.DMA((N_FWD,)),
            pltpu.SemaphoreType.DMA((N_BWD,)),
            pltpu.SemaphoreType.DMA((N_BWD,)),
        ],
    )(x2, Wq, k2, v2, Wo)
    return out.reshape(1, SQ, D)
